# baseline (speedup 1.0000x reference)
import functools

import jax
import jax.numpy as jnp
from jax import lax
from jax.experimental import pallas as pl
from jax.experimental.pallas import tpu as pltpu
from jax.experimental.pallas import tpu_sc as plsc

N = 10000
D = 128
NP = 10240
E = 320000
CHUNK = 128
NCHUNK = 80
NBUF = 2
NSB = 2 * NBUF
EPT = NCHUNK * CHUNK
EPAD = 32 * EPT
NCH0 = 160
ROWS_PER_SUBCORE = NP // 16
BR = 512
NB = NP // BR


def _mesh():
    return plsc.VectorSubcoreMesh(core_axis_name="c", subcore_axis_name="s")


def _sc_deg_body(dst_hbm, ones_hbm, zeros_hbm, out_hbm, idxd_v, ones_v, deg_sh):
    c = lax.axis_index("c")
    s = lax.axis_index("s")
    wid = s * 2 + c
    pltpu.sync_copy(zeros_hbm.at[pl.ds(s * ROWS_PER_SUBCORE, ROWS_PER_SUBCORE)],
                    deg_sh.at[pl.ds(s * ROWS_PER_SUBCORE, ROWS_PER_SUBCORE)])
    pltpu.sync_copy(ones_hbm, ones_v)
    pltpu.sync_copy(dst_hbm.at[wid], idxd_v)
    plsc.subcore_barrier()

    def body(i, carry):
        pltpu.sync_copy(ones_v, deg_sh.at[idxd_v.at[i]], add=True)
        return carry

    lax.fori_loop(0, NCHUNK, body, 0)
    plsc.subcore_barrier()
    base = c * NP + s * ROWS_PER_SUBCORE
    pltpu.sync_copy(deg_sh.at[pl.ds(s * ROWS_PER_SUBCORE, ROWS_PER_SUBCORE)],
                    out_hbm.at[pl.ds(base, ROWS_PER_SUBCORE)])


def _sc_deg(dst3, ones_d, zerosd):
    k = functools.partial(
        pl.kernel,
        mesh=_mesh(),
        out_type=jax.ShapeDtypeStruct((2 * NP, D), jnp.float32),
        scratch_types=[
            pltpu.VMEM((NCHUNK, CHUNK), jnp.int32),
            pltpu.VMEM((CHUNK, D), jnp.float32),
            pltpu.VMEM_SHARED((NP, D), jnp.float32),
        ],
    )(_sc_deg_body)
    return k(dst3, ones_d, zerosd)


def _sc_msg_body(src2, dst2, hp_hbm, zeros_hbm, out_hbm,
                 i0, i1, i2, i3, d0, d1, d2, d3, r0, r1, acc_sh,
                 gs0, gs1, ss0, ss1, ss2, ss3, ds0, ds1, ds2, ds3):
    rows = (r0, r1)
    gsem = (gs0, gs1)
    sidx = (i0, i1, i2, i3)
    ssem = (ss0, ss1, ss2, ss3)
    didx = (d0, d1, d2, d3)
    dsem = (ds0, ds1, ds2, ds3)
    c = lax.axis_index("c")
    s = lax.axis_index("s")
    rb = pl.ds(s * ROWS_PER_SUBCORE, ROWS_PER_SUBCORE)
    pltpu.sync_copy(zeros_hbm.at[rb], acc_sh.at[rb])
    nch = jnp.where(c == 0, NCH0, 0)
    cb = jnp.where(c == 0, s * NCH0, 0)

    for j in range(NSB):
        pltpu.async_copy(src2.at[cb + j], sidx[j], ssem[j])
        pltpu.async_copy(dst2.at[cb + j], didx[j], dsem[j])
    for b in range(NBUF):
        pltpu.make_async_copy(src2.at[cb + b], sidx[b], ssem[b]).wait()
        pltpu.async_copy(hp_hbm.at[sidx[b]], rows[b], gsem[b])
    plsc.subcore_barrier()

    def outer(o, carry):
        for j in range(NSB):
            b = j % NBUF
            g = o * NSB + j
            pltpu.make_async_copy(hp_hbm.at[sidx[j]], rows[b], gsem[b]).wait()
            pltpu.make_async_copy(dst2.at[cb + g], didx[j], dsem[j]).wait()
            pltpu.sync_copy(rows[b], acc_sh.at[didx[j]], add=True)
            nxt_load = g + NSB
            nxt_gather = g + NBUF

            @pl.when(nxt_load < nch)
            def _():
                pltpu.async_copy(src2.at[cb + nxt_load], sidx[j], ssem[j])
                pltpu.async_copy(dst2.at[cb + nxt_load], didx[j], dsem[j])

            @pl.when(nxt_gather < nch)
            def _():
                j2 = (j + NBUF) % NSB
                pltpu.make_async_copy(src2.at[cb + nxt_gather], sidx[j2],
                                      ssem[j2]).wait()
                pltpu.async_copy(hp_hbm.at[sidx[j2]], rows[b], gsem[b])
        return carry

    lax.fori_loop(0, nch // NSB, outer, 0)

    @pl.when(c != 0)
    def _():
        for j in range(NSB):
            pltpu.make_async_copy(dst2.at[cb + j], didx[j], dsem[j]).wait()
        for j in range(NBUF, NSB):
            pltpu.make_async_copy(src2.at[cb + j], sidx[j], ssem[j]).wait()
        for b in range(NBUF):
            pltpu.make_async_copy(hp_hbm.at[sidx[b]], rows[b], gsem[b]).wait()

    plsc.subcore_barrier()

    @pl.when(c == 0)
    def _():
        pltpu.sync_copy(acc_sh.at[rb], out_hbm.at[rb])


def _sc_msg(src2, dst2, hp, zerosd):
    k = functools.partial(
        pl.kernel,
        mesh=_mesh(),
        out_type=jax.ShapeDtypeStruct((NP, D), jnp.float32),
        scratch_types=(
            [pltpu.VMEM((CHUNK,), jnp.int32) for _ in range(2 * NSB)]
            + [pltpu.VMEM((CHUNK, D), jnp.float32) for _ in range(NBUF)]
            + [pltpu.VMEM_SHARED((NP, D), jnp.float32)]
            + [pltpu.SemaphoreType.DMA for _ in range(NBUF + 2 * NSB)]
        ),
    )(_sc_msg_body)
    return k(src2, dst2, hp, zerosd)


def _tc1_body(deg_ref, x_ref, w_ref, h_ref, dis_ref):
    dg = deg_ref[...]
    degsum = dg[0, :, 0:1] + dg[1, :, 0:1] + 1.0
    disb = lax.rsqrt(degsum)
    h = jnp.dot(x_ref[...], w_ref[...], preferred_element_type=jnp.float32)
    h_ref[...] = h * disb
    dis_ref[...] = disb


def _tc1(deg2, xp, w1):
    return pl.pallas_call(
        _tc1_body,
        grid=(NB,),
        in_specs=[
            pl.BlockSpec((2, BR, D), lambda i: (0, i, 0)),
            pl.BlockSpec((BR, D), lambda i: (i, 0)),
            pl.BlockSpec((D, D), lambda i: (0, 0)),
        ],
        out_specs=[
            pl.BlockSpec((BR, D), lambda i: (i, 0)),
            pl.BlockSpec((BR, 1), lambda i: (i, 0)),
        ],
        out_shape=[
            jax.ShapeDtypeStruct((NP, D), jnp.float32),
            jax.ShapeDtypeStruct((NP, 1), jnp.float32),
        ],
    )(deg2, xp, w1)


def _tca_body(agg_ref, h_ref, dis_ref, b_ref, t_ref, stats_ref):
    i = pl.program_id(0)
    t = (agg_ref[...] + h_ref[...]) * dis_ref[...] + b_ref[...]
    row_ids = lax.broadcasted_iota(jnp.int32, (BR, 1), 0) + i * BR
    mask = row_ids < N
    tm = jnp.where(mask, t, 0.0)
    s1 = jnp.sum(tm, axis=0, keepdims=True)
    s2 = jnp.sum(tm * tm, axis=0, keepdims=True)
    t_ref[...] = t

    @pl.when(i == 0)
    def _():
        stats_ref[...] = jnp.zeros((8, D), jnp.float32)

    stats_ref[0:1, :] += s1
    stats_ref[1:2, :] += s2


def _tca(agg, hp, dis, b):
    return pl.pallas_call(
        _tca_body,
        grid=(NB,),
        in_specs=[
            pl.BlockSpec((BR, D), lambda i: (i, 0)),
            pl.BlockSpec((BR, D), lambda i: (i, 0)),
            pl.BlockSpec((BR, 1), lambda i: (i, 0)),
            pl.BlockSpec((1, D), lambda i: (0, 0)),
        ],
        out_specs=[
            pl.BlockSpec((BR, D), lambda i: (i, 0)),
            pl.BlockSpec((8, D), lambda i: (0, 0)),
        ],
        out_shape=[
            jax.ShapeDtypeStruct((NP, D), jnp.float32),
            jax.ShapeDtypeStruct((8, D), jnp.float32),
        ],
    )(agg, hp, dis, b)


def _tcb_body(t_ref, stats_ref, dis_ref, g_ref, be_ref, w_ref, h_ref):
    st = stats_ref[...]
    mean = st[0:1, :] * (1.0 / N)
    var = st[1:2, :] * (1.0 / N) - mean * mean
    scale = lax.rsqrt(var + 1e-5) * g_ref[...]
    y = jnp.maximum((t_ref[...] - mean) * scale + be_ref[...], 0.0)
    h = jnp.dot(y, w_ref[...], preferred_element_type=jnp.float32)
    h_ref[...] = h * dis_ref[...]


def _tcb(t, stats, dis, g, be, w):
    return pl.pallas_call(
        _tcb_body,
        grid=(NB,),
        in_specs=[
            pl.BlockSpec((BR, D), lambda i: (i, 0)),
            pl.BlockSpec((8, D), lambda i: (0, 0)),
            pl.BlockSpec((BR, 1), lambda i: (i, 0)),
            pl.BlockSpec((1, D), lambda i: (0, 0)),
            pl.BlockSpec((1, D), lambda i: (0, 0)),
            pl.BlockSpec((D, D), lambda i: (0, 0)),
        ],
        out_specs=pl.BlockSpec((BR, D), lambda i: (i, 0)),
        out_shape=jax.ShapeDtypeStruct((NP, D), jnp.float32),
    )(t, stats, dis, g, be, w)


def _tc_final_body(agg_ref, h_ref, dis_ref, b_ref, o_ref):
    o_ref[...] = (agg_ref[...] + h_ref[...]) * dis_ref[...] + b_ref[...]


def _tc_final(agg, hp, dis, b):
    return pl.pallas_call(
        _tc_final_body,
        grid=(NB,),
        in_specs=[
            pl.BlockSpec((BR, D), lambda i: (i, 0)),
            pl.BlockSpec((BR, D), lambda i: (i, 0)),
            pl.BlockSpec((BR, 1), lambda i: (i, 0)),
            pl.BlockSpec((1, D), lambda i: (0, 0)),
        ],
        out_specs=pl.BlockSpec((BR, D), lambda i: (i, 0)),
        out_shape=jax.ShapeDtypeStruct((NP, D), jnp.float32),
    )(agg, hp, dis, b)


def kernel(x, edge_index, W1, b1, g1, be1, W2, b2, g2, be2, W3, b3):
    pad = EPAD - E
    src = jnp.concatenate([edge_index[0], jnp.full((pad,), N, jnp.int32)])
    dst = jnp.concatenate([edge_index[1], jnp.full((pad,), N, jnp.int32)])
    src3 = src.reshape(32, NCHUNK, CHUNK)
    dst3 = dst.reshape(32, NCHUNK, CHUNK)
    src2 = src.reshape(EPAD // CHUNK, CHUNK)
    dst2 = dst.reshape(EPAD // CHUNK, CHUNK)
    xp = jnp.zeros((NP, D), jnp.float32).at[:N].set(x)

    ones_d = jnp.ones((CHUNK, D), jnp.float32)
    zerosd = jnp.zeros((NP, D), jnp.float32)
    b1r = b1.reshape(1, D)
    b2r = b2.reshape(1, D)
    b3r = b3.reshape(1, D)
    g1r = g1.reshape(1, D)
    g2r = g2.reshape(1, D)
    be1r = be1.reshape(1, D)
    be2r = be2.reshape(1, D)

    deg2 = _sc_deg(dst3, ones_d, zerosd).reshape(2, NP, D)
    h1p, dis = _tc1(deg2, xp, W1)

    agg1 = _sc_msg(src2, dst2, h1p, zerosd)
    t1, st1 = _tca(agg1, h1p, dis, b1r)
    h2p = _tcb(t1, st1, dis, g1r, be1r, W2)

    agg2 = _sc_msg(src2, dst2, h2p, zerosd)
    t2, st2 = _tca(agg2, h2p, dis, b2r)
    h3p = _tcb(t2, st2, dis, g2r, be2r, W3)

    agg3 = _sc_msg(src2, dst2, h3p, zerosd)
    out = _tc_final(agg3, h3p, dis, b3r)
    return out[:N]


# speedup vs baseline: 1.1080x; 1.1080x over previous
import functools

import jax
import jax.numpy as jnp
from jax import lax
from jax.experimental import pallas as pl
from jax.experimental.pallas import tpu as pltpu
from jax.experimental.pallas import tpu_sc as plsc

N = 10000
D = 128
NP = 10240
E = 320000
CHUNK = 128
NCHUNK = 80
NBUF = 2
NSB = 2 * NBUF
EPT = NCHUNK * CHUNK
EPAD = 32 * EPT
NCH0 = 120
NCH1 = 40
ROWS_PER_SUBCORE = NP // 16
BR = 512
NB = NP // BR


def _mesh():
    return plsc.VectorSubcoreMesh(core_axis_name="c", subcore_axis_name="s")


def _sc_deg_body(dst_hbm, ones_hbm, zeros_hbm, out_hbm, idxd_v, ones_v, deg_sh):
    c = lax.axis_index("c")
    s = lax.axis_index("s")
    wid = s * 2 + c
    pltpu.sync_copy(zeros_hbm.at[pl.ds(s * ROWS_PER_SUBCORE, ROWS_PER_SUBCORE)],
                    deg_sh.at[pl.ds(s * ROWS_PER_SUBCORE, ROWS_PER_SUBCORE)])
    pltpu.sync_copy(ones_hbm, ones_v)
    pltpu.sync_copy(dst_hbm.at[wid], idxd_v)
    plsc.subcore_barrier()

    def body(i, carry):
        pltpu.sync_copy(ones_v, deg_sh.at[idxd_v.at[i]], add=True)
        return carry

    lax.fori_loop(0, NCHUNK, body, 0)
    plsc.subcore_barrier()
    base = c * NP + s * ROWS_PER_SUBCORE
    pltpu.sync_copy(deg_sh.at[pl.ds(s * ROWS_PER_SUBCORE, ROWS_PER_SUBCORE)],
                    out_hbm.at[pl.ds(base, ROWS_PER_SUBCORE)])


def _sc_deg(dst3, ones_d, zerosd):
    k = functools.partial(
        pl.kernel,
        mesh=_mesh(),
        out_type=jax.ShapeDtypeStruct((2 * NP, D), jnp.float32),
        scratch_types=[
            pltpu.VMEM((NCHUNK, CHUNK), jnp.int32),
            pltpu.VMEM((CHUNK, D), jnp.float32),
            pltpu.VMEM_SHARED((NP, D), jnp.float32),
        ],
    )(_sc_deg_body)
    return k(dst3, ones_d, zerosd)


def _sc_msg_body(src2, dst2, hp_hbm, zeros_hbm, out_hbm,
                 i0, i1, i2, i3, d0, d1, d2, d3, sidx2, didx2, r0, r1, acc_sh,
                 gs0, gs1, ss0, ss1, ss2, ss3, ds0, ds1, ds2, ds3):
    rows = (r0, r1)
    gsem = (gs0, gs1)
    sidx = (i0, i1, i2, i3)
    ssem = (ss0, ss1, ss2, ss3)
    didx = (d0, d1, d2, d3)
    dsem = (ds0, ds1, ds2, ds3)
    c = lax.axis_index("c")
    s = lax.axis_index("s")
    rb = pl.ds(s * ROWS_PER_SUBCORE, ROWS_PER_SUBCORE)
    pltpu.sync_copy(zeros_hbm.at[rb], acc_sh.at[rb])
    cb = s * NCH0
    cb1 = 16 * NCH0 + s * NCH1

    @pl.when(c == 0)
    def _():
        for j in range(NSB):
            pltpu.async_copy(src2.at[cb + j], sidx[j], ssem[j])
            pltpu.async_copy(dst2.at[cb + j], didx[j], dsem[j])
        for b in range(NBUF):
            pltpu.make_async_copy(src2.at[cb + b], sidx[b], ssem[b]).wait()
            pltpu.async_copy(hp_hbm.at[sidx[b]], rows[b], gsem[b])

    @pl.when(c == 1)
    def _():
        pltpu.sync_copy(src2.at[pl.ds(cb1, NCH1)], sidx2)
        pltpu.sync_copy(dst2.at[pl.ds(cb1, NCH1)], didx2)

    plsc.subcore_barrier()

    def outer(o, carry):
        for j in range(NSB):
            b = j % NBUF
            g = o * NSB + j
            pltpu.make_async_copy(hp_hbm.at[sidx[j]], rows[b], gsem[b]).wait()
            pltpu.make_async_copy(dst2.at[cb + g], didx[j], dsem[j]).wait()
            pltpu.sync_copy(rows[b], acc_sh.at[didx[j]], add=True)
            nxt_load = g + NSB
            nxt_gather = g + NBUF

            @pl.when(nxt_load < NCH0)
            def _():
                pltpu.async_copy(src2.at[cb + nxt_load], sidx[j], ssem[j])
                pltpu.async_copy(dst2.at[cb + nxt_load], didx[j], dsem[j])

            @pl.when(nxt_gather < NCH0)
            def _():
                j2 = (j + NBUF) % NSB
                pltpu.make_async_copy(src2.at[cb + nxt_gather], sidx[j2],
                                      ssem[j2]).wait()
                pltpu.async_copy(hp_hbm.at[sidx[j2]], rows[b], gsem[b])
        return carry

    def serial(i, carry):
        pltpu.async_copy(hp_hbm.at[sidx2.at[i]], rows[0], gsem[0])
        pltpu.make_async_copy(hp_hbm.at[sidx2.at[i]], rows[0], gsem[0]).wait()
        pltpu.sync_copy(rows[0], acc_sh.at[didx2.at[i]], add=True)
        return carry

    @pl.when(c == 0)
    def _():
        lax.fori_loop(0, NCH0 // NSB, outer, 0)

    @pl.when(c == 1)
    def _():
        lax.fori_loop(0, NCH1, serial, 0)

    plsc.subcore_barrier()
    base = c * NP + s * ROWS_PER_SUBCORE
    pltpu.sync_copy(acc_sh.at[rb], out_hbm.at[pl.ds(base, ROWS_PER_SUBCORE)])


def _sc_msg(src2, dst2, hp, zerosd):
    k = functools.partial(
        pl.kernel,
        mesh=_mesh(),
        out_type=jax.ShapeDtypeStruct((2 * NP, D), jnp.float32),
        scratch_types=(
            [pltpu.VMEM((CHUNK,), jnp.int32) for _ in range(2 * NSB)]
            + [pltpu.VMEM((NCH1, CHUNK), jnp.int32) for _ in range(2)]
            + [pltpu.VMEM((CHUNK, D), jnp.float32) for _ in range(NBUF)]
            + [pltpu.VMEM_SHARED((NP, D), jnp.float32)]
            + [pltpu.SemaphoreType.DMA for _ in range(NBUF + 2 * NSB)]
        ),
    )(_sc_msg_body)
    return k(src2, dst2, hp, zerosd)


def _tc1_body(deg_ref, x_ref, w_ref, h_ref, dis_ref):
    dg = deg_ref[...]
    degsum = dg[0, :, 0:1] + dg[1, :, 0:1] + 1.0
    disb = lax.rsqrt(degsum)
    h = jnp.dot(x_ref[...], w_ref[...], preferred_element_type=jnp.float32)
    h_ref[...] = h * disb
    dis_ref[...] = disb


def _tc1(deg2, xp, w1):
    return pl.pallas_call(
        _tc1_body,
        grid=(NB,),
        in_specs=[
            pl.BlockSpec((2, BR, D), lambda i: (0, i, 0)),
            pl.BlockSpec((BR, D), lambda i: (i, 0)),
            pl.BlockSpec((D, D), lambda i: (0, 0)),
        ],
        out_specs=[
            pl.BlockSpec((BR, D), lambda i: (i, 0)),
            pl.BlockSpec((BR, 1), lambda i: (i, 0)),
        ],
        out_shape=[
            jax.ShapeDtypeStruct((NP, D), jnp.float32),
            jax.ShapeDtypeStruct((NP, 1), jnp.float32),
        ],
    )(deg2, xp, w1)


def _tca_body(agg_ref, h_ref, dis_ref, b_ref, t_ref, stats_ref):
    i = pl.program_id(0)
    a = agg_ref[...]
    t = (a[0] + a[1] + h_ref[...]) * dis_ref[...] + b_ref[...]
    row_ids = lax.broadcasted_iota(jnp.int32, (BR, 1), 0) + i * BR
    mask = row_ids < N
    tm = jnp.where(mask, t, 0.0)
    s1 = jnp.sum(tm, axis=0, keepdims=True)
    s2 = jnp.sum(tm * tm, axis=0, keepdims=True)
    t_ref[...] = t

    @pl.when(i == 0)
    def _():
        stats_ref[...] = jnp.zeros((8, D), jnp.float32)

    stats_ref[0:1, :] += s1
    stats_ref[1:2, :] += s2


def _tca(agg, hp, dis, b):
    return pl.pallas_call(
        _tca_body,
        grid=(NB,),
        in_specs=[
            pl.BlockSpec((2, BR, D), lambda i: (0, i, 0)),
            pl.BlockSpec((BR, D), lambda i: (i, 0)),
            pl.BlockSpec((BR, 1), lambda i: (i, 0)),
            pl.BlockSpec((1, D), lambda i: (0, 0)),
        ],
        out_specs=[
            pl.BlockSpec((BR, D), lambda i: (i, 0)),
            pl.BlockSpec((8, D), lambda i: (0, 0)),
        ],
        out_shape=[
            jax.ShapeDtypeStruct((NP, D), jnp.float32),
            jax.ShapeDtypeStruct((8, D), jnp.float32),
        ],
    )(agg, hp, dis, b)


def _tcb_body(t_ref, stats_ref, dis_ref, g_ref, be_ref, w_ref, h_ref):
    st = stats_ref[...]
    mean = st[0:1, :] * (1.0 / N)
    var = st[1:2, :] * (1.0 / N) - mean * mean
    scale = lax.rsqrt(var + 1e-5) * g_ref[...]
    y = jnp.maximum((t_ref[...] - mean) * scale + be_ref[...], 0.0)
    h = jnp.dot(y, w_ref[...], preferred_element_type=jnp.float32)
    h_ref[...] = h * dis_ref[...]


def _tcb(t, stats, dis, g, be, w):
    return pl.pallas_call(
        _tcb_body,
        grid=(NB,),
        in_specs=[
            pl.BlockSpec((BR, D), lambda i: (i, 0)),
            pl.BlockSpec((8, D), lambda i: (0, 0)),
            pl.BlockSpec((BR, 1), lambda i: (i, 0)),
            pl.BlockSpec((1, D), lambda i: (0, 0)),
            pl.BlockSpec((1, D), lambda i: (0, 0)),
            pl.BlockSpec((D, D), lambda i: (0, 0)),
        ],
        out_specs=pl.BlockSpec((BR, D), lambda i: (i, 0)),
        out_shape=jax.ShapeDtypeStruct((NP, D), jnp.float32),
    )(t, stats, dis, g, be, w)


def _tc_final_body(agg_ref, h_ref, dis_ref, b_ref, o_ref):
    a = agg_ref[...]
    o_ref[...] = (a[0] + a[1] + h_ref[...]) * dis_ref[...] + b_ref[...]


def _tc_final(agg, hp, dis, b):
    return pl.pallas_call(
        _tc_final_body,
        grid=(NB,),
        in_specs=[
            pl.BlockSpec((2, BR, D), lambda i: (0, i, 0)),
            pl.BlockSpec((BR, D), lambda i: (i, 0)),
            pl.BlockSpec((BR, 1), lambda i: (i, 0)),
            pl.BlockSpec((1, D), lambda i: (0, 0)),
        ],
        out_specs=pl.BlockSpec((BR, D), lambda i: (i, 0)),
        out_shape=jax.ShapeDtypeStruct((NP, D), jnp.float32),
    )(agg, hp, dis, b)


def kernel(x, edge_index, W1, b1, g1, be1, W2, b2, g2, be2, W3, b3):
    pad = EPAD - E
    src = jnp.concatenate([edge_index[0], jnp.full((pad,), N, jnp.int32)])
    dst = jnp.concatenate([edge_index[1], jnp.full((pad,), N, jnp.int32)])
    src3 = src.reshape(32, NCHUNK, CHUNK)
    dst3 = dst.reshape(32, NCHUNK, CHUNK)
    src2 = src.reshape(EPAD // CHUNK, CHUNK)
    dst2 = dst.reshape(EPAD // CHUNK, CHUNK)
    xp = jnp.zeros((NP, D), jnp.float32).at[:N].set(x)

    ones_d = jnp.ones((CHUNK, D), jnp.float32)
    zerosd = jnp.zeros((NP, D), jnp.float32)
    b1r = b1.reshape(1, D)
    b2r = b2.reshape(1, D)
    b3r = b3.reshape(1, D)
    g1r = g1.reshape(1, D)
    g2r = g2.reshape(1, D)
    be1r = be1.reshape(1, D)
    be2r = be2.reshape(1, D)

    deg2 = _sc_deg(dst3, ones_d, zerosd).reshape(2, NP, D)
    h1p, dis = _tc1(deg2, xp, W1)

    agg1 = _sc_msg(src2, dst2, h1p, zerosd).reshape(2, NP, D)
    t1, st1 = _tca(agg1, h1p, dis, b1r)
    h2p = _tcb(t1, st1, dis, g1r, be1r, W2)

    agg2 = _sc_msg(src2, dst2, h2p, zerosd).reshape(2, NP, D)
    t2, st2 = _tca(agg2, h2p, dis, b2r)
    h3p = _tcb(t2, st2, dis, g2r, be2r, W3)

    agg3 = _sc_msg(src2, dst2, h3p, zerosd).reshape(2, NP, D)
    out = _tc_final(agg3, h3p, dis, b3r)
    return out[:N]


# speedup vs baseline: 1.5958x; 1.4402x over previous
import functools

import jax
import jax.numpy as jnp
from jax import lax
from jax.experimental import pallas as pl
from jax.experimental.pallas import tpu as pltpu
from jax.experimental.pallas import tpu_sc as plsc

N = 10000
D = 128
NP = 10240
E = 320000
CHUNK = 128
NCHUNK = 79
EPT = NCHUNK * CHUNK
EPAD = 32 * EPT
ROWS_PER_SUBCORE = NP // 16
BR = 512
NB = NP // BR


def _mesh():
    return plsc.VectorSubcoreMesh(core_axis_name="c", subcore_axis_name="s")


def _sc_deg_body(dst_hbm, ones_hbm, zeros_hbm, out_hbm, idxd_v, ones_v, deg_sh):
    c = lax.axis_index("c")
    s = lax.axis_index("s")
    wid = s * 2 + c
    pltpu.sync_copy(zeros_hbm.at[pl.ds(s * ROWS_PER_SUBCORE, ROWS_PER_SUBCORE)],
                    deg_sh.at[pl.ds(s * ROWS_PER_SUBCORE, ROWS_PER_SUBCORE)])
    pltpu.sync_copy(ones_hbm, ones_v)
    pltpu.sync_copy(dst_hbm.at[wid], idxd_v)
    plsc.subcore_barrier()

    def body(i, carry):
        pltpu.sync_copy(ones_v, deg_sh.at[idxd_v.at[i]], add=True)
        return carry

    lax.fori_loop(0, NCHUNK, body, 0)
    plsc.subcore_barrier()
    base = c * NP + s * ROWS_PER_SUBCORE
    pltpu.sync_copy(deg_sh.at[pl.ds(s * ROWS_PER_SUBCORE, ROWS_PER_SUBCORE)],
                    out_hbm.at[pl.ds(base, ROWS_PER_SUBCORE)])


def _sc_deg(dst3, ones_d, zerosd):
    k = functools.partial(
        pl.kernel,
        mesh=_mesh(),
        out_type=jax.ShapeDtypeStruct((2 * NP, D), jnp.float32),
        scratch_types=[
            pltpu.VMEM((NCHUNK, CHUNK), jnp.int32),
            pltpu.VMEM((CHUNK, D), jnp.float32),
            pltpu.VMEM_SHARED((NP, D), jnp.float32),
        ],
    )(_sc_deg_body)
    return k(dst3, ones_d, zerosd)


def _sc_msg_body(src3, dst3, h_hbm, zeros_hbm, out_hbm,
                 idxs_v, idxd_v, rows_v, acc_sh, sem):
    c = lax.axis_index("c")
    s = lax.axis_index("s")
    wid = s * 2 + c
    pltpu.sync_copy(zeros_hbm.at[pl.ds(s * ROWS_PER_SUBCORE, ROWS_PER_SUBCORE)],
                    acc_sh.at[pl.ds(s * ROWS_PER_SUBCORE, ROWS_PER_SUBCORE)])
    pltpu.sync_copy(src3.at[wid], idxs_v)
    pltpu.sync_copy(dst3.at[wid], idxd_v)
    plsc.subcore_barrier()

    def body(i, carry):
        pltpu.async_copy(h_hbm.at[idxs_v.at[i]], rows_v, sem).wait()
        pltpu.sync_copy(rows_v, acc_sh.at[idxd_v.at[i]], add=True)
        return carry

    lax.fori_loop(0, NCHUNK, body, 0)
    plsc.subcore_barrier()
    base = c * NP + s * ROWS_PER_SUBCORE
    pltpu.sync_copy(acc_sh.at[pl.ds(s * ROWS_PER_SUBCORE, ROWS_PER_SUBCORE)],
                    out_hbm.at[pl.ds(base, ROWS_PER_SUBCORE)])


def _sc_msg(src3, dst3, hp, zerosd):
    k = functools.partial(
        pl.kernel,
        mesh=_mesh(),
        out_type=jax.ShapeDtypeStruct((2 * NP, D), jnp.float32),
        scratch_types=[
            pltpu.VMEM((NCHUNK, CHUNK), jnp.int32),
            pltpu.VMEM((NCHUNK, CHUNK), jnp.int32),
            pltpu.VMEM((CHUNK, D), jnp.float32),
            pltpu.VMEM_SHARED((NP, D), jnp.float32),
            pltpu.SemaphoreType.DMA,
        ],
    )(_sc_msg_body)
    return k(src3, dst3, hp, zerosd)


def _tc1_body(deg_ref, x_ref, w_ref, h_ref, dis_ref):
    dg = deg_ref[...]
    degsum = dg[0, :, 0:1] + dg[1, :, 0:1] + 1.0
    disb = lax.rsqrt(degsum)
    h = jnp.dot(x_ref[...], w_ref[...], preferred_element_type=jnp.float32)
    h_ref[...] = h * disb
    dis_ref[...] = disb


def _tc1(deg2, xp, w1):
    return pl.pallas_call(
        _tc1_body,
        grid=(NB,),
        in_specs=[
            pl.BlockSpec((2, BR, D), lambda i: (0, i, 0)),
            pl.BlockSpec((BR, D), lambda i: (i, 0)),
            pl.BlockSpec((D, D), lambda i: (0, 0)),
        ],
        out_specs=[
            pl.BlockSpec((BR, D), lambda i: (i, 0)),
            pl.BlockSpec((BR, 1), lambda i: (i, 0)),
        ],
        out_shape=[
            jax.ShapeDtypeStruct((NP, D), jnp.float32),
            jax.ShapeDtypeStruct((NP, 1), jnp.float32),
        ],
    )(deg2, xp, w1)


def _tca_body(agg_ref, h_ref, dis_ref, b_ref, t_ref, stats_ref):
    i = pl.program_id(0)
    a = agg_ref[...]
    t = (a[0] + a[1] + h_ref[...]) * dis_ref[...] + b_ref[...]
    row_ids = lax.broadcasted_iota(jnp.int32, (BR, 1), 0) + i * BR
    mask = row_ids < N
    tm = jnp.where(mask, t, 0.0)
    s1 = jnp.sum(tm, axis=0, keepdims=True)
    s2 = jnp.sum(tm * tm, axis=0, keepdims=True)
    t_ref[...] = t

    @pl.when(i == 0)
    def _():
        stats_ref[...] = jnp.zeros((8, D), jnp.float32)

    stats_ref[0:1, :] += s1
    stats_ref[1:2, :] += s2


def _tca(agg, hp, dis, b):
    return pl.pallas_call(
        _tca_body,
        grid=(NB,),
        in_specs=[
            pl.BlockSpec((2, BR, D), lambda i: (0, i, 0)),
            pl.BlockSpec((BR, D), lambda i: (i, 0)),
            pl.BlockSpec((BR, 1), lambda i: (i, 0)),
            pl.BlockSpec((1, D), lambda i: (0, 0)),
        ],
        out_specs=[
            pl.BlockSpec((BR, D), lambda i: (i, 0)),
            pl.BlockSpec((8, D), lambda i: (0, 0)),
        ],
        out_shape=[
            jax.ShapeDtypeStruct((NP, D), jnp.float32),
            jax.ShapeDtypeStruct((8, D), jnp.float32),
        ],
    )(agg, hp, dis, b)


def _tcb_body(t_ref, stats_ref, dis_ref, g_ref, be_ref, w_ref, h_ref):
    st = stats_ref[...]
    mean = st[0:1, :] * (1.0 / N)
    var = st[1:2, :] * (1.0 / N) - mean * mean
    scale = lax.rsqrt(var + 1e-5) * g_ref[...]
    y = jnp.maximum((t_ref[...] - mean) * scale + be_ref[...], 0.0)
    h = jnp.dot(y, w_ref[...], preferred_element_type=jnp.float32)
    h_ref[...] = h * dis_ref[...]


def _tcb(t, stats, dis, g, be, w):
    return pl.pallas_call(
        _tcb_body,
        grid=(NB,),
        in_specs=[
            pl.BlockSpec((BR, D), lambda i: (i, 0)),
            pl.BlockSpec((8, D), lambda i: (0, 0)),
            pl.BlockSpec((BR, 1), lambda i: (i, 0)),
            pl.BlockSpec((1, D), lambda i: (0, 0)),
            pl.BlockSpec((1, D), lambda i: (0, 0)),
            pl.BlockSpec((D, D), lambda i: (0, 0)),
        ],
        out_specs=pl.BlockSpec((BR, D), lambda i: (i, 0)),
        out_shape=jax.ShapeDtypeStruct((NP, D), jnp.float32),
    )(t, stats, dis, g, be, w)


def _tc_final_body(agg_ref, h_ref, dis_ref, b_ref, o_ref):
    a = agg_ref[...]
    o_ref[...] = (a[0] + a[1] + h_ref[...]) * dis_ref[...] + b_ref[...]


def _tc_final(agg, hp, dis, b):
    return pl.pallas_call(
        _tc_final_body,
        grid=(NB,),
        in_specs=[
            pl.BlockSpec((2, BR, D), lambda i: (0, i, 0)),
            pl.BlockSpec((BR, D), lambda i: (i, 0)),
            pl.BlockSpec((BR, 1), lambda i: (i, 0)),
            pl.BlockSpec((1, D), lambda i: (0, 0)),
        ],
        out_specs=pl.BlockSpec((BR, D), lambda i: (i, 0)),
        out_shape=jax.ShapeDtypeStruct((NP, D), jnp.float32),
    )(agg, hp, dis, b)


def kernel(x, edge_index, W1, b1, g1, be1, W2, b2, g2, be2, W3, b3):
    pad = EPAD - E
    src = jnp.concatenate([edge_index[0], jnp.full((pad,), N, jnp.int32)])
    dst = jnp.concatenate([edge_index[1], jnp.full((pad,), N, jnp.int32)])
    src3 = src.reshape(32, NCHUNK, CHUNK)
    dst3 = dst.reshape(32, NCHUNK, CHUNK)
    xp = jnp.zeros((NP, D), jnp.float32).at[:N].set(x)

    ones_d = jnp.ones((CHUNK, D), jnp.float32)
    zerosd = jnp.zeros((NP, D), jnp.float32)
    b1r = b1.reshape(1, D)
    b2r = b2.reshape(1, D)
    b3r = b3.reshape(1, D)
    g1r = g1.reshape(1, D)
    g2r = g2.reshape(1, D)
    be1r = be1.reshape(1, D)
    be2r = be2.reshape(1, D)

    deg2 = _sc_deg(dst3, ones_d, zerosd).reshape(2, NP, D)
    h1p, dis = _tc1(deg2, xp, W1)

    agg1 = _sc_msg(src3, dst3, h1p, zerosd).reshape(2, NP, D)
    t1, st1 = _tca(agg1, h1p, dis, b1r)
    h2p = _tcb(t1, st1, dis, g1r, be1r, W2)

    agg2 = _sc_msg(src3, dst3, h2p, zerosd).reshape(2, NP, D)
    t2, st2 = _tca(agg2, h2p, dis, b2r)
    h3p = _tcb(t2, st2, dis, g2r, be2r, W3)

    agg3 = _sc_msg(src3, dst3, h3p, zerosd).reshape(2, NP, D)
    out = _tc_final(agg3, h3p, dis, b3r)
    return out[:N]
